# baseline (speedup 1.0000x reference)
import functools

import jax
import jax.numpy as jnp
from jax.experimental import pallas as pl
from jax.experimental.pallas import tpu as pltpu

DIM_C = 96


def _cov_kernel(label_ref, org_ref, aug_ref, out_ref, *, n_b, n_hc, hw):
    b = pl.program_id(0)
    hc = pl.program_id(1)

    @pl.when(jnp.logical_and(b == 0, hc == 0))
    def _init():
        out_ref[...] = jnp.zeros_like(out_ref)

    minv = label_ref[0]
    for i in range(1, n_b):
        minv = jnp.minimum(minv, label_ref[i])
    is_low = (label_ref[b] == minv).astype(jnp.float32)

    scale = 1.0 / (hw - 1)
    w_low = is_low * scale
    w_high = scale - w_low

    c_dim, h_dim, w_dim = org_ref.shape[1:]
    xo = org_ref[0].astype(jnp.bfloat16).reshape(c_dim, h_dim * w_dim)
    xa = aug_ref[0].astype(jnp.bfloat16).reshape(c_dim, h_dim * w_dim)
    dn = (((1,), (1,)), ((), ()))
    po = jax.lax.dot_general(xo, xo, dn, preferred_element_type=jnp.float32)
    pa = jax.lax.dot_general(xa, xa, dn, preferred_element_type=jnp.float32)

    out_ref[0, 0] += w_low * po
    out_ref[0, 1] += w_low * pa
    out_ref[1, 0] += w_high * po
    out_ref[1, 1] += w_high * pa

    @pl.when(jnp.logical_and(b == n_b - 1, hc == n_hc - 1))
    def _finish():
        row = jax.lax.broadcasted_iota(jnp.int32, (DIM_C, DIM_C), 0)
        col = jax.lax.broadcasted_iota(jnp.int32, (DIM_C, DIM_C), 1)
        mask = (row < col).astype(jnp.float32)
        out_ref[...] = out_ref[...] * mask[None, None]


def kernel(org_input, aug_input, contrast_label):
    b, c, h, w = org_input.shape
    hw = h * w

    n_hc = 2
    grid = (b, n_hc)
    in_spec = pl.BlockSpec((1, c, h // n_hc, w), lambda i, j: (i, 0, j, 0))
    out = pl.pallas_call(
        functools.partial(_cov_kernel, n_b=b, n_hc=n_hc, hw=hw),
        grid=grid,
        in_specs=[
            pl.BlockSpec(memory_space=pltpu.SMEM),
            in_spec,
            in_spec,
        ],
        out_specs=pl.BlockSpec((2, 2, c, c), lambda i, j: (0, 0, 0, 0)),
        out_shape=jax.ShapeDtypeStruct((2, 2, c, c), jnp.float32),
    )(contrast_label, org_input, aug_input)
    return out


# speedup vs baseline: 1.0757x; 1.0757x over previous
import functools

import jax
import jax.numpy as jnp
from jax.experimental import pallas as pl
from jax.experimental.pallas import tpu as pltpu

DIM_C = 96


def _cov_kernel(label_ref, org_ref, aug_ref, out_ref, *, n_b, n_j, hw):
    i = pl.program_id(0)
    j = pl.program_id(1)

    @pl.when(j == 0)
    def _init():
        out_ref[...] = jnp.zeros_like(out_ref)

    minv = label_ref[0]
    for s in range(1, n_b):
        minv = jnp.minimum(minv, label_ref[s])
    is_low = (label_ref[i * n_j + j] == minv).astype(jnp.float32)

    scale = 1.0 / (hw - 1)
    w_low = is_low * scale
    w_high = scale - w_low

    c_dim, h_dim, w_dim = org_ref.shape[1:]
    xo = org_ref[0].astype(jnp.bfloat16).reshape(c_dim, h_dim * w_dim)
    xa = aug_ref[0].astype(jnp.bfloat16).reshape(c_dim, h_dim * w_dim)
    dn = (((1,), (1,)), ((), ()))
    po = jax.lax.dot_general(xo, xo, dn, preferred_element_type=jnp.float32)
    pa = jax.lax.dot_general(xa, xa, dn, preferred_element_type=jnp.float32)

    out_ref[0, 0, 0] += w_low * po
    out_ref[0, 0, 1] += w_low * pa
    out_ref[0, 1, 0] += w_high * po
    out_ref[0, 1, 1] += w_high * pa

    @pl.when(j == n_j - 1)
    def _finish():
        row = jax.lax.broadcasted_iota(jnp.int32, (DIM_C, DIM_C), 0)
        col = jax.lax.broadcasted_iota(jnp.int32, (DIM_C, DIM_C), 1)
        mask = (row < col).astype(jnp.float32)
        out_ref[...] = out_ref[...] * mask[None, None, None]


def kernel(org_input, aug_input, contrast_label):
    b, c, h, w = org_input.shape
    hw = h * w
    n_cores = 2
    n_j = b // n_cores

    grid = (n_cores, n_j)
    in_spec = pl.BlockSpec((1, c, h, w), lambda i, j: (i * n_j + j, 0, 0, 0))
    partials = pl.pallas_call(
        functools.partial(_cov_kernel, n_b=b, n_j=n_j, hw=hw),
        grid=grid,
        in_specs=[
            pl.BlockSpec(memory_space=pltpu.SMEM),
            in_spec,
            in_spec,
        ],
        out_specs=pl.BlockSpec((1, 2, 2, c, c), lambda i, j: (i, 0, 0, 0, 0)),
        out_shape=jax.ShapeDtypeStruct((n_cores, 2, 2, c, c), jnp.float32),
        compiler_params=pltpu.CompilerParams(
            dimension_semantics=("parallel", "arbitrary")),
    )(contrast_label, org_input, aug_input)
    return partials[0] + partials[1]


# speedup vs baseline: 1.1251x; 1.0459x over previous
import functools

import jax
import jax.numpy as jnp
from jax.experimental import pallas as pl
from jax.experimental.pallas import tpu as pltpu

DIM_C = 96


def _cov_kernel(label_ref, org_ref, aug_ref, out_ref, *, n_b, hw):
    b = pl.program_id(0)

    @pl.when(b == 0)
    def _init():
        out_ref[...] = jnp.zeros_like(out_ref)

    minv = label_ref[0]
    for i in range(1, n_b):
        minv = jnp.minimum(minv, label_ref[i])
    is_low = (label_ref[b] == minv).astype(jnp.float32)

    scale = 1.0 / (hw - 1)
    w_low = is_low * scale
    w_high = scale - w_low

    c_dim, h_dim, w_dim = org_ref.shape[1:]
    xo = org_ref[0].astype(jnp.bfloat16).reshape(c_dim, h_dim * w_dim)
    xa = aug_ref[0].astype(jnp.bfloat16).reshape(c_dim, h_dim * w_dim)
    dn = (((1,), (1,)), ((), ()))
    po = jax.lax.dot_general(xo, xo, dn, preferred_element_type=jnp.float32)
    pa = jax.lax.dot_general(xa, xa, dn, preferred_element_type=jnp.float32)

    out_ref[0, 0] += w_low * po
    out_ref[0, 1] += w_low * pa
    out_ref[1, 0] += w_high * po
    out_ref[1, 1] += w_high * pa

    @pl.when(b == n_b - 1)
    def _finish():
        row = jax.lax.broadcasted_iota(jnp.int32, (DIM_C, DIM_C), 0)
        col = jax.lax.broadcasted_iota(jnp.int32, (DIM_C, DIM_C), 1)
        mask = (row < col).astype(jnp.float32)
        out_ref[...] = out_ref[...] * mask[None, None]


def kernel(org_input, aug_input, contrast_label):
    b, c, h, w = org_input.shape
    hw = h * w

    grid = (b,)
    in_spec = pl.BlockSpec((1, c, h, w), lambda i: (i, 0, 0, 0))
    out = pl.pallas_call(
        functools.partial(_cov_kernel, n_b=b, hw=hw),
        grid=grid,
        in_specs=[
            pl.BlockSpec(memory_space=pltpu.SMEM),
            in_spec,
            in_spec,
        ],
        out_specs=pl.BlockSpec((2, 2, c, c), lambda i: (0, 0, 0, 0)),
        out_shape=jax.ShapeDtypeStruct((2, 2, c, c), jnp.float32),
    )(contrast_label, org_input, aug_input)
    return out
